# Initial kernel scaffold; baseline (speedup 1.0000x reference)
import jax
import jax.numpy as jnp
from jax.experimental import pallas as pl


def kernel(feat, offsets, W1, b1, g1, be1, W2, b2, g2, be2, W3, b3):
    raise NotImplementedError("write your pallas kernel here")


# speedup vs baseline: 10.3093x; 10.3093x over previous
import functools
import math

import jax
import jax.numpy as jnp
from jax.experimental import pallas as pl
from jax.experimental.pallas import tpu as pltpu

B = 16
N = 32768
D = 256
H1 = 512
H2 = 256
C = 50

ROWS_PER_BLOCK = 2048


def _segsum_body(lo_ref, hi_ref, feat_ref, out_ref):
    k = pl.program_id(0)

    @pl.when(k == 0)
    def _():
        out_ref[...] = jnp.zeros_like(out_ref)

    row0 = k * ROWS_PER_BLOCK
    rows = row0 + jax.lax.broadcasted_iota(jnp.int32, (1, ROWS_PER_BLOCK), 1)
    lo = lo_ref[...]
    hi = hi_ref[...]
    mask = jnp.logical_and(rows >= lo, rows < hi).astype(jnp.float32)
    out_ref[...] += jnp.dot(mask, feat_ref[...],
                            preferred_element_type=jnp.float32)


def _erf(x):
    a1, a2, a3, a4, a5 = (0.254829592, -0.284496736, 1.421413741,
                          -1.453152027, 1.061405429)
    p = 0.3275911
    ax = jnp.abs(x)
    t = 1.0 / (1.0 + p * ax)
    poly = t * (a1 + t * (a2 + t * (a3 + t * (a4 + t * a5))))
    y = 1.0 - poly * jnp.exp(-ax * ax)
    return jnp.sign(x) * y


def _gelu(x):
    return 0.5 * x * (1.0 + _erf(x * (1.0 / math.sqrt(2.0))))


def _layernorm(x, g, b, eps=1e-5):
    m = jnp.mean(x, axis=-1, keepdims=True)
    v = jnp.mean((x - m) ** 2, axis=-1, keepdims=True)
    return (x - m) * jax.lax.rsqrt(v + eps) * g + b


def _mlp_body(sums_ref, counts_ref, W1_ref, b1_ref, g1_ref, be1_ref,
              W2_ref, b2_ref, g2_ref, be2_ref, W3_ref, b3_ref, out_ref):
    means = sums_ref[...] / jnp.maximum(counts_ref[...], 1.0)
    h = jnp.dot(means, W1_ref[...], preferred_element_type=jnp.float32)
    h = h + b1_ref[...]
    h = _layernorm(h, g1_ref[...], be1_ref[...])
    h = _gelu(h)
    h = jnp.dot(h, W2_ref[...], preferred_element_type=jnp.float32)
    h = h + b2_ref[...]
    h = _layernorm(h, g2_ref[...], be2_ref[...])
    h = _gelu(h)
    out = jnp.dot(h, W3_ref[...], preferred_element_type=jnp.float32)
    out_ref[...] = out + b3_ref[...]


@jax.jit
def kernel(feat, offsets, W1, b1, g1, be1, W2, b2, g2, be2, W3, b3):
    off = offsets.astype(jnp.int32)
    lo = off[:-1].reshape(B, 1)
    hi = off[1:].reshape(B, 1)
    counts = (hi - lo).astype(jnp.float32)

    num_blocks = N // ROWS_PER_BLOCK
    sums = pl.pallas_call(
        _segsum_body,
        grid=(num_blocks,),
        in_specs=[
            pl.BlockSpec((B, 1), lambda k: (0, 0)),
            pl.BlockSpec((B, 1), lambda k: (0, 0)),
            pl.BlockSpec((ROWS_PER_BLOCK, D), lambda k: (k, 0)),
        ],
        out_specs=pl.BlockSpec((B, D), lambda k: (0, 0)),
        out_shape=jax.ShapeDtypeStruct((B, D), jnp.float32),
    )(lo, hi, feat)

    out = pl.pallas_call(
        _mlp_body,
        in_specs=[
            pl.BlockSpec((B, D), lambda: (0, 0)),
            pl.BlockSpec((B, 1), lambda: (0, 0)),
            pl.BlockSpec((D, H1), lambda: (0, 0)),
            pl.BlockSpec((1, H1), lambda: (0, 0)),
            pl.BlockSpec((1, H1), lambda: (0, 0)),
            pl.BlockSpec((1, H1), lambda: (0, 0)),
            pl.BlockSpec((H1, H2), lambda: (0, 0)),
            pl.BlockSpec((1, H2), lambda: (0, 0)),
            pl.BlockSpec((1, H2), lambda: (0, 0)),
            pl.BlockSpec((1, H2), lambda: (0, 0)),
            pl.BlockSpec((H2, C), lambda: (0, 0)),
            pl.BlockSpec((1, C), lambda: (0, 0)),
        ],
        out_specs=pl.BlockSpec((B, C), lambda: (0, 0)),
        out_shape=jax.ShapeDtypeStruct((B, C), jnp.float32),
    )(sums, counts, W1, b1.reshape(1, H1), g1.reshape(1, H1),
      be1.reshape(1, H1), W2, b2.reshape(1, H2), g2.reshape(1, H2),
      be2.reshape(1, H2), W3, b3.reshape(1, C))
    return out
